# baseline (speedup 1.0000x reference)
import functools

import jax
import jax.numpy as jnp
from jax import lax
from jax.experimental import pallas as pl
from jax.experimental.pallas import tpu as pltpu
from jax.experimental.pallas import tpu_sc as plsc

N = 10000
N_PAD = 10240
F_IN = 128
HID = 128
C_OUT = 17
C_PAD = 128
E = 320000
E_PAD = 327680
NW = 32
ROWS_W = E_PAD // NW // 128
BLK = 8
ROWS_TOT = E_PAD // 128
ROWS_C0 = 2560
ROWS_C1 = ROWS_TOT - ROWS_C0
RPS = N_PAD // 16
R_TC = 1280
G_TC = N_PAD // R_TC


def _sc_mesh():
    return plsc.VectorSubcoreMesh(core_axis_name="c", subcore_axis_name="s",
                                  num_cores=2, num_subcores=16)


def _make_deg():
    @functools.partial(
        pl.kernel,
        out_type=jax.ShapeDtypeStruct((2, N_PAD, 128), jnp.float32),
        mesh=_sc_mesh(),
        scratch_types=[
            pltpu.VMEM((BLK, 128), jnp.int32),
            pltpu.VMEM((128, 128), jnp.float32),
            pltpu.VMEM_SHARED((N_PAD, 128), jnp.float32),
            pltpu.SemaphoreType.DMA,
        ],
    )
    def deg_kernel(dst_hbm, zeros_hbm, ones_hbm, out_hbm, dst_v, ones_v, acc_sh,
                   sem):
        c = lax.axis_index("c")
        s = lax.axis_index("s")
        wid = c * 16 + s
        pltpu.sync_copy(zeros_hbm.at[pl.ds(s * RPS, RPS)],
                        acc_sh.at[pl.ds(s * RPS, RPS)])
        pltpu.sync_copy(ones_hbm, ones_v)
        plsc.subcore_barrier()
        row0 = wid * ROWS_W

        def outer(g, carry):
            pltpu.sync_copy(dst_hbm.at[pl.ds(row0 + g * BLK, BLK)], dst_v)
            cps = [pltpu.async_copy(ones_v, acc_sh.at[dst_v.at[j]], sem, add=True)
                   for j in range(BLK)]
            for cp in cps:
                cp.wait()
            return carry

        lax.fori_loop(0, ROWS_W // BLK, outer, 0)
        plsc.subcore_barrier()
        pltpu.sync_copy(acc_sh.at[pl.ds(s * RPS, RPS)],
                        out_hbm.at[c, pl.ds(s * RPS, RPS)])

    return deg_kernel


def _make_agg():
    @functools.partial(
        pl.kernel,
        out_type=jax.ShapeDtypeStruct((2, N_PAD, 128), jnp.float32),
        mesh=_sc_mesh(),
        scratch_types=[
            pltpu.VMEM((BLK, 128), jnp.int32),
            pltpu.VMEM((BLK, 128), jnp.int32),
            pltpu.VMEM((128, 128), jnp.float32),
            pltpu.VMEM((128, 128), jnp.float32),
            pltpu.VMEM_SHARED((N_PAD, 128), jnp.float32),
            pltpu.SemaphoreType.DMA,
            pltpu.SemaphoreType.DMA,
        ],
    )
    def agg_kernel(y_hbm, src_hbm, dst_hbm, zeros_hbm, out_hbm,
                   src_v, dst_v, rows_a, rows_b, acc_sh, sem_a, sem_b):
        c = lax.axis_index("c")
        s = lax.axis_index("s")
        wid = c * 16 + s
        sl = pl.ds(s * RPS, RPS)
        pltpu.sync_copy(zeros_hbm.at[sl], acc_sh.at[sl])
        plsc.subcore_barrier()
        is0 = c == 0
        row0 = jnp.where(is0, s * (ROWS_C0 // 16), ROWS_C0 + s * (ROWS_C1 // 16))
        nblk = jnp.where(is0, ROWS_C0 // 16 // BLK, ROWS_C1 // 16 // BLK)
        bufs = ((rows_a, sem_a), (rows_b, sem_b))

        def outer(g, carry):
            pltpu.sync_copy(src_hbm.at[pl.ds(row0 + g * BLK, BLK)], src_v)
            pltpu.sync_copy(dst_hbm.at[pl.ds(row0 + g * BLK, BLK)], dst_v)
            pltpu.async_copy(y_hbm.at[src_v.at[0]], rows_a, sem_a)
            for j in range(BLK):
                buf, sem = bufs[j % 2]
                pltpu.make_async_copy(y_hbm.at[src_v.at[j]], buf, sem).wait()
                if j + 1 < BLK:
                    nbuf, nsem = bufs[(j + 1) % 2]
                    pltpu.async_copy(y_hbm.at[src_v.at[j + 1]], nbuf, nsem)
                pltpu.sync_copy(buf, acc_sh.at[dst_v.at[j]], add=True)
            return carry

        lax.fori_loop(0, nblk, outer, 0)
        plsc.subcore_barrier()
        pltpu.sync_copy(acc_sh.at[sl], out_hbm.at[c, sl])

    return agg_kernel


_make_deg = functools.lru_cache(maxsize=None)(_make_deg)
_make_agg = functools.lru_cache(maxsize=None)(_make_agg)


def _dis_invdeg(deg_ref):
    deg = 1.0 + deg_ref[0][:, 0:1] + deg_ref[1][:, 0:1]
    return lax.rsqrt(deg), 1.0 / deg


def _stage1_body(x_ref, w_ref, deg_ref, y_ref, xw_ref):
    xw = jnp.dot(x_ref[...], w_ref[...], preferred_element_type=jnp.float32)
    dis, _ = _dis_invdeg(deg_ref)
    xw_ref[...] = xw
    y_ref[...] = xw * dis


def _stage2_body(a_ref, xw_ref, deg_ref, b1_ref, w2_ref, y2_ref, xw2_ref):
    dis, inv_deg = _dis_invdeg(deg_ref)
    o1 = (a_ref[0] + a_ref[1]) * dis + xw_ref[...] * inv_deg + b1_ref[...]
    h = jnp.maximum(o1, 0.0)
    xw2 = jnp.dot(h, w2_ref[...], preferred_element_type=jnp.float32)
    xw2_ref[...] = xw2
    y2_ref[...] = xw2 * dis


def _stage3_body(a_ref, xw2_ref, deg_ref, b2_ref, out_ref):
    dis, inv_deg = _dis_invdeg(deg_ref)
    o = (a_ref[0] + a_ref[1]) * dis + xw2_ref[...] * inv_deg + b2_ref[...]
    logits = o[:, :C_OUT]
    m = jnp.max(logits, axis=1, keepdims=True)
    lse = m + jnp.log(jnp.sum(jnp.exp(logits - m), axis=1, keepdims=True))
    out_ref[...] = logits - lse


def _row_spec(w):
    return pl.BlockSpec((R_TC, w), lambda i: (i, 0))


def _part_spec(w):
    return pl.BlockSpec((2, R_TC, w), lambda i: (0, i, 0))


def _full_spec(r, w):
    return pl.BlockSpec((r, w), lambda i: (0, 0))


def _stage1(x_p, w1, degp):
    return pl.pallas_call(
        _stage1_body,
        grid=(G_TC,),
        in_specs=[_row_spec(F_IN), _full_spec(F_IN, HID), _part_spec(128)],
        out_specs=[_row_spec(HID), _row_spec(HID)],
        out_shape=[jax.ShapeDtypeStruct((N_PAD, HID), jnp.float32),
                   jax.ShapeDtypeStruct((N_PAD, HID), jnp.float32)],
    )(x_p, w1, degp)


def _stage2(acc1, xw1, degp, b1, w2p):
    return pl.pallas_call(
        _stage2_body,
        grid=(G_TC,),
        in_specs=[_part_spec(HID), _row_spec(HID), _part_spec(128),
                  _full_spec(1, HID), _full_spec(HID, C_PAD)],
        out_specs=[_row_spec(C_PAD), _row_spec(C_PAD)],
        out_shape=[jax.ShapeDtypeStruct((N_PAD, C_PAD), jnp.float32),
                   jax.ShapeDtypeStruct((N_PAD, C_PAD), jnp.float32)],
    )(acc1, xw1, degp, b1, w2p)


def _stage3(acc2, xw2, degp, b2p):
    return pl.pallas_call(
        _stage3_body,
        grid=(G_TC,),
        in_specs=[_part_spec(C_PAD), _row_spec(C_PAD), _part_spec(128),
                  _full_spec(1, C_PAD)],
        out_specs=_row_spec(C_OUT),
        out_shape=jax.ShapeDtypeStruct((N_PAD, C_OUT), jnp.float32),
    )(acc2, xw2, degp, b2p)


def kernel(x, edge_index, W1, b1, W2, b2):
    idt = edge_index.dtype
    pad_idx = jnp.full((E_PAD - E,), N, dtype=idt)
    srcp = jnp.concatenate([edge_index[0], pad_idx]).reshape(E_PAD // 128, 128)
    dstp = jnp.concatenate([edge_index[1], pad_idx]).reshape(E_PAD // 128, 128)
    srcp = srcp.astype(jnp.int32)
    dstp = dstp.astype(jnp.int32)

    x_p = jnp.pad(x, ((0, N_PAD - N), (0, 0)))
    w2p = jnp.pad(W2, ((0, 0), (0, C_PAD - C_OUT)))
    b1r = b1.reshape(1, HID)
    b2p = jnp.pad(b2, (0, C_PAD - C_OUT)).reshape(1, C_PAD)
    zeros128 = jnp.zeros((N_PAD, HID), jnp.float32)
    ones128 = jnp.ones((128, 128), jnp.float32)

    degp = _make_deg()(dstp, zeros128, ones128)
    y1, xw1 = _stage1(x_p, W1, degp)
    acc1 = _make_agg()(y1, srcp, dstp, zeros128)
    y2, xw2 = _stage2(acc1, xw1, degp, b1r, w2p)
    acc2 = _make_agg()(y2, srcp, dstp, zeros128)
    outp = _stage3(acc2, xw2, degp, b2p)
    return outp[:N]


# speedup vs baseline: 1.3324x; 1.3324x over previous
import functools

import jax
import jax.numpy as jnp
from jax import lax
from jax.experimental import pallas as pl
from jax.experimental.pallas import tpu as pltpu
from jax.experimental.pallas import tpu_sc as plsc

N = 10000
N_PAD = 10240
F_IN = 128
HID = 128
C_OUT = 17
C_PAD = 128
E = 320000
E_PAD = 327680
NW = 32
ROWS_W = E_PAD // NW // 128
BLK = 8
ROWS_TOT = E_PAD // 128
ROWS_C0 = 2048
ROWS_C1 = ROWS_TOT - ROWS_C0
RPS = N_PAD // 16
R_TC = 1280
G_TC = N_PAD // R_TC


def _sc_mesh():
    return plsc.VectorSubcoreMesh(core_axis_name="c", subcore_axis_name="s",
                                  num_cores=2, num_subcores=16)


def _make_deg():
    @functools.partial(
        pl.kernel,
        out_type=jax.ShapeDtypeStruct((2, N_PAD, 128), jnp.float32),
        mesh=_sc_mesh(),
        scratch_types=[
            pltpu.VMEM((BLK, 128), jnp.int32),
            pltpu.VMEM((128, 128), jnp.float32),
            pltpu.VMEM_SHARED((N_PAD, 128), jnp.float32),
            pltpu.SemaphoreType.DMA,
        ],
    )
    def deg_kernel(dst_hbm, zeros_hbm, ones_hbm, out_hbm, dst_v, ones_v, acc_sh,
                   sem):
        c = lax.axis_index("c")
        s = lax.axis_index("s")
        wid = c * 16 + s
        pltpu.sync_copy(zeros_hbm.at[pl.ds(s * RPS, RPS)],
                        acc_sh.at[pl.ds(s * RPS, RPS)])
        pltpu.sync_copy(ones_hbm, ones_v)
        plsc.subcore_barrier()
        row0 = wid * ROWS_W

        def outer(g, carry):
            pltpu.sync_copy(dst_hbm.at[pl.ds(row0 + g * BLK, BLK)], dst_v)
            cps = [pltpu.async_copy(ones_v, acc_sh.at[dst_v.at[j]], sem, add=True)
                   for j in range(BLK)]
            for cp in cps:
                cp.wait()
            return carry

        lax.fori_loop(0, ROWS_W // BLK, outer, 0)
        plsc.subcore_barrier()
        pltpu.sync_copy(acc_sh.at[pl.ds(s * RPS, RPS)],
                        out_hbm.at[c, pl.ds(s * RPS, RPS)])

    return deg_kernel


def _make_agg():
    @functools.partial(
        pl.kernel,
        out_type=jax.ShapeDtypeStruct((2, N_PAD, 128), jnp.float32),
        mesh=_sc_mesh(),
        scratch_types=[
            pltpu.VMEM((BLK, 128), jnp.int32),
            pltpu.VMEM((BLK, 128), jnp.int32),
            pltpu.VMEM((128, 128), jnp.float32),
            pltpu.VMEM((128, 128), jnp.float32),
            pltpu.VMEM_SHARED((N_PAD, 128), jnp.float32),
            pltpu.SemaphoreType.DMA,
            pltpu.SemaphoreType.DMA,
        ],
    )
    def agg_kernel(y_hbm, src_hbm, dst_hbm, zeros_hbm, out_hbm,
                   src_v, dst_v, rows_a, rows_b, acc_sh, sem_a, sem_b):
        c = lax.axis_index("c")
        s = lax.axis_index("s")
        wid = c * 16 + s
        sl = pl.ds(s * RPS, RPS)
        pltpu.sync_copy(zeros_hbm.at[sl], acc_sh.at[sl])
        plsc.subcore_barrier()
        is0 = c == 0
        row0 = jnp.where(is0, s * (ROWS_C0 // 16), ROWS_C0 + s * (ROWS_C1 // 16))
        nblk = jnp.where(is0, ROWS_C0 // 16 // BLK, ROWS_C1 // 16 // BLK)
        bufs = ((rows_a, sem_a), (rows_b, sem_b))

        def outer(g, carry):
            pltpu.sync_copy(src_hbm.at[pl.ds(row0 + g * BLK, BLK)], src_v)
            pltpu.sync_copy(dst_hbm.at[pl.ds(row0 + g * BLK, BLK)], dst_v)
            pltpu.async_copy(y_hbm.at[src_v.at[0]], rows_a, sem_a)
            for j in range(BLK):
                buf, sem = bufs[j % 2]
                pltpu.make_async_copy(y_hbm.at[src_v.at[j]], buf, sem).wait()
                if j + 1 < BLK:
                    nbuf, nsem = bufs[(j + 1) % 2]
                    pltpu.async_copy(y_hbm.at[src_v.at[j + 1]], nbuf, nsem)
                pltpu.sync_copy(buf, acc_sh.at[dst_v.at[j]], add=True)
            return carry

        lax.fori_loop(0, nblk, outer, 0)
        plsc.subcore_barrier()
        pltpu.sync_copy(acc_sh.at[sl], out_hbm.at[c, sl])

    return agg_kernel


_make_deg = functools.lru_cache(maxsize=None)(_make_deg)
_make_agg = functools.lru_cache(maxsize=None)(_make_agg)


def _dis_invdeg(deg_ref):
    deg = 1.0 + deg_ref[0][:, 0:1] + deg_ref[1][:, 0:1]
    return lax.rsqrt(deg), 1.0 / deg


def _stage1_body(x_ref, w_ref, deg_ref, y_ref, xw_ref):
    xw = jnp.dot(x_ref[...], w_ref[...], preferred_element_type=jnp.float32)
    dis, _ = _dis_invdeg(deg_ref)
    xw_ref[...] = xw
    y_ref[...] = xw * dis


def _stage2_body(a_ref, xw_ref, deg_ref, b1_ref, w2_ref, y2_ref, xw2_ref):
    dis, inv_deg = _dis_invdeg(deg_ref)
    o1 = (a_ref[0] + a_ref[1]) * dis + xw_ref[...] * inv_deg + b1_ref[...]
    h = jnp.maximum(o1, 0.0)
    xw2 = jnp.dot(h, w2_ref[...], preferred_element_type=jnp.float32)
    xw2_ref[...] = xw2
    y2_ref[...] = xw2 * dis


def _stage3_body(a_ref, xw2_ref, deg_ref, b2_ref, out_ref):
    dis, inv_deg = _dis_invdeg(deg_ref)
    o = (a_ref[0] + a_ref[1]) * dis + xw2_ref[...] * inv_deg + b2_ref[...]
    logits = o[:, :C_OUT]
    m = jnp.max(logits, axis=1, keepdims=True)
    lse = m + jnp.log(jnp.sum(jnp.exp(logits - m), axis=1, keepdims=True))
    out_ref[...] = logits - lse


def _row_spec(w):
    return pl.BlockSpec((R_TC, w), lambda i: (i, 0))


def _part_spec(w):
    return pl.BlockSpec((2, R_TC, w), lambda i: (0, i, 0))


def _full_spec(r, w):
    return pl.BlockSpec((r, w), lambda i: (0, 0))


def _stage1(x_p, w1, degp):
    return pl.pallas_call(
        _stage1_body,
        grid=(G_TC,),
        in_specs=[_row_spec(F_IN), _full_spec(F_IN, HID), _part_spec(128)],
        out_specs=[_row_spec(HID), _row_spec(HID)],
        out_shape=[jax.ShapeDtypeStruct((N_PAD, HID), jnp.float32),
                   jax.ShapeDtypeStruct((N_PAD, HID), jnp.float32)],
    )(x_p, w1, degp)


def _stage2(acc1, xw1, degp, b1, w2p):
    return pl.pallas_call(
        _stage2_body,
        grid=(G_TC,),
        in_specs=[_part_spec(HID), _row_spec(HID), _part_spec(128),
                  _full_spec(1, HID), _full_spec(HID, C_PAD)],
        out_specs=[_row_spec(C_PAD), _row_spec(C_PAD)],
        out_shape=[jax.ShapeDtypeStruct((N_PAD, C_PAD), jnp.float32),
                   jax.ShapeDtypeStruct((N_PAD, C_PAD), jnp.float32)],
    )(acc1, xw1, degp, b1, w2p)


def _stage3(acc2, xw2, degp, b2p):
    return pl.pallas_call(
        _stage3_body,
        grid=(G_TC,),
        in_specs=[_part_spec(C_PAD), _row_spec(C_PAD), _part_spec(128),
                  _full_spec(1, C_PAD)],
        out_specs=_row_spec(C_OUT),
        out_shape=jax.ShapeDtypeStruct((N_PAD, C_OUT), jnp.float32),
    )(acc2, xw2, degp, b2p)


def kernel(x, edge_index, W1, b1, W2, b2):
    idt = edge_index.dtype
    pad_idx = jnp.full((E_PAD - E,), N, dtype=idt)
    srcp = jnp.concatenate([edge_index[0], pad_idx]).reshape(E_PAD // 128, 128)
    dstp = jnp.concatenate([edge_index[1], pad_idx]).reshape(E_PAD // 128, 128)
    srcp = srcp.astype(jnp.int32)
    dstp = dstp.astype(jnp.int32)

    x_p = jnp.pad(x, ((0, N_PAD - N), (0, 0)))
    w2p = jnp.pad(W2, ((0, 0), (0, C_PAD - C_OUT)))
    b1r = b1.reshape(1, HID)
    b2p = jnp.pad(b2, (0, C_PAD - C_OUT)).reshape(1, C_PAD)
    zeros128 = jnp.zeros((N_PAD, HID), jnp.float32)
    ones128 = jnp.ones((128, 128), jnp.float32)

    degp = _make_deg()(dstp, zeros128, ones128)
    y1, xw1 = _stage1(x_p, W1, degp)
    acc1 = _make_agg()(y1, srcp, dstp, zeros128)
    y2, xw2 = _stage2(acc1, xw1, degp, b1r, w2p)
    acc2 = _make_agg()(y2, srcp, dstp, zeros128)
    outp = _stage3(acc2, xw2, degp, b2p)
    return outp[:N]


# speedup vs baseline: 1.6643x; 1.2491x over previous
import functools

import jax
import jax.numpy as jnp
from jax import lax
from jax.experimental import pallas as pl
from jax.experimental.pallas import tpu as pltpu
from jax.experimental.pallas import tpu_sc as plsc

N = 10000
N_PAD = 10240
F_IN = 128
HID = 128
C_OUT = 17
C_PAD = 128
E = 320000
E_PAD = 327680
NW = 32
ROWS_W = E_PAD // NW // 128
BLK = 8
ROWS_TOT = E_PAD // 128
ROWS_C0 = 2304
ROWS_C1 = ROWS_TOT - ROWS_C0
RPS = N_PAD // 16
R_TC = 1280
G_TC = N_PAD // R_TC


def _sc_mesh():
    return plsc.VectorSubcoreMesh(core_axis_name="c", subcore_axis_name="s",
                                  num_cores=2, num_subcores=16)


def _make_deg():
    @functools.partial(
        pl.kernel,
        out_type=jax.ShapeDtypeStruct((2, N_PAD, 128), jnp.float32),
        mesh=_sc_mesh(),
        scratch_types=[
            pltpu.VMEM((BLK, 128), jnp.int32),
            pltpu.VMEM((128, 128), jnp.float32),
            pltpu.VMEM_SHARED((N_PAD, 128), jnp.float32),
            pltpu.SemaphoreType.DMA,
        ],
    )
    def deg_kernel(dst_hbm, zeros_hbm, ones_hbm, out_hbm, dst_v, ones_v, acc_sh,
                   sem):
        c = lax.axis_index("c")
        s = lax.axis_index("s")
        wid = c * 16 + s
        pltpu.sync_copy(zeros_hbm.at[pl.ds(s * RPS, RPS)],
                        acc_sh.at[pl.ds(s * RPS, RPS)])
        pltpu.sync_copy(ones_hbm, ones_v)
        plsc.subcore_barrier()
        row0 = wid * ROWS_W

        def outer(g, carry):
            pltpu.sync_copy(dst_hbm.at[pl.ds(row0 + g * BLK, BLK)], dst_v)
            cps = [pltpu.async_copy(ones_v, acc_sh.at[dst_v.at[j]], sem, add=True)
                   for j in range(BLK)]
            for cp in cps:
                cp.wait()
            return carry

        lax.fori_loop(0, ROWS_W // BLK, outer, 0)
        plsc.subcore_barrier()
        pltpu.sync_copy(acc_sh.at[pl.ds(s * RPS, RPS)],
                        out_hbm.at[c, pl.ds(s * RPS, RPS)])

    return deg_kernel


def _make_agg():
    @functools.partial(
        pl.kernel,
        out_type=jax.ShapeDtypeStruct((2, N_PAD, 128), jnp.float32),
        mesh=_sc_mesh(),
        scratch_types=[
            pltpu.VMEM((BLK, 128), jnp.int32),
            pltpu.VMEM((BLK, 128), jnp.int32),
            pltpu.VMEM((128, 128), jnp.float32),
            pltpu.VMEM((128, 128), jnp.float32),
            pltpu.VMEM_SHARED((N_PAD, 128), jnp.float32),
            pltpu.SemaphoreType.DMA,
            pltpu.SemaphoreType.DMA,
        ],
    )
    def agg_kernel(y_hbm, src_hbm, dst_hbm, zeros_hbm, out_hbm,
                   src_v, dst_v, rows_a, rows_b, acc_sh, sem_a, sem_b):
        c = lax.axis_index("c")
        s = lax.axis_index("s")
        wid = c * 16 + s
        sl = pl.ds(s * RPS, RPS)
        pltpu.sync_copy(zeros_hbm.at[sl], acc_sh.at[sl])
        plsc.subcore_barrier()
        is0 = c == 0
        row0 = jnp.where(is0, s * (ROWS_C0 // 16), ROWS_C0 + s * (ROWS_C1 // 16))
        nblk = jnp.where(is0, ROWS_C0 // 16 // BLK, ROWS_C1 // 16 // BLK)
        bufs = ((rows_a, sem_a), (rows_b, sem_b))

        def outer(g, carry):
            pltpu.sync_copy(src_hbm.at[pl.ds(row0 + g * BLK, BLK)], src_v)
            pltpu.sync_copy(dst_hbm.at[pl.ds(row0 + g * BLK, BLK)], dst_v)
            pltpu.async_copy(y_hbm.at[src_v.at[0]], rows_a, sem_a)
            for j in range(BLK):
                buf, sem = bufs[j % 2]
                pltpu.make_async_copy(y_hbm.at[src_v.at[j]], buf, sem).wait()
                if j + 1 < BLK:
                    nbuf, nsem = bufs[(j + 1) % 2]
                    pltpu.async_copy(y_hbm.at[src_v.at[j + 1]], nbuf, nsem)
                pltpu.sync_copy(buf, acc_sh.at[dst_v.at[j]], add=True)
            return carry

        lax.fori_loop(0, nblk, outer, 0)
        plsc.subcore_barrier()
        pltpu.sync_copy(acc_sh.at[sl], out_hbm.at[c, sl])

    return agg_kernel


_make_deg = functools.lru_cache(maxsize=None)(_make_deg)
_make_agg = functools.lru_cache(maxsize=None)(_make_agg)


def _dis_invdeg(deg_ref):
    deg = 1.0 + deg_ref[0][:, 0:1] + deg_ref[1][:, 0:1]
    return lax.rsqrt(deg), 1.0 / deg


def _stage1_body(x_ref, w_ref, deg_ref, y_ref, xw_ref):
    xw = jnp.dot(x_ref[...], w_ref[...], preferred_element_type=jnp.float32)
    dis, _ = _dis_invdeg(deg_ref)
    xw_ref[...] = xw
    y_ref[...] = xw * dis


def _stage2_body(a_ref, xw_ref, deg_ref, b1_ref, w2_ref, y2_ref, xw2_ref):
    dis, inv_deg = _dis_invdeg(deg_ref)
    o1 = (a_ref[0] + a_ref[1]) * dis + xw_ref[...] * inv_deg + b1_ref[...]
    h = jnp.maximum(o1, 0.0)
    xw2 = jnp.dot(h, w2_ref[...], preferred_element_type=jnp.float32)
    xw2_ref[...] = xw2
    y2_ref[...] = xw2 * dis


def _stage3_body(a_ref, xw2_ref, deg_ref, b2_ref, out_ref):
    dis, inv_deg = _dis_invdeg(deg_ref)
    o = (a_ref[0] + a_ref[1]) * dis + xw2_ref[...] * inv_deg + b2_ref[...]
    logits = o[:, :C_OUT]
    m = jnp.max(logits, axis=1, keepdims=True)
    lse = m + jnp.log(jnp.sum(jnp.exp(logits - m), axis=1, keepdims=True))
    out_ref[...] = logits - lse


def _row_spec(w):
    return pl.BlockSpec((R_TC, w), lambda i: (i, 0))


def _part_spec(w):
    return pl.BlockSpec((2, R_TC, w), lambda i: (0, i, 0))


def _full_spec(r, w):
    return pl.BlockSpec((r, w), lambda i: (0, 0))


def _stage1(x_p, w1, degp):
    return pl.pallas_call(
        _stage1_body,
        grid=(G_TC,),
        in_specs=[_row_spec(F_IN), _full_spec(F_IN, HID), _part_spec(128)],
        out_specs=[_row_spec(HID), _row_spec(HID)],
        out_shape=[jax.ShapeDtypeStruct((N_PAD, HID), jnp.float32),
                   jax.ShapeDtypeStruct((N_PAD, HID), jnp.float32)],
    )(x_p, w1, degp)


def _stage2(acc1, xw1, degp, b1, w2p):
    return pl.pallas_call(
        _stage2_body,
        grid=(G_TC,),
        in_specs=[_part_spec(HID), _row_spec(HID), _part_spec(128),
                  _full_spec(1, HID), _full_spec(HID, C_PAD)],
        out_specs=[_row_spec(C_PAD), _row_spec(C_PAD)],
        out_shape=[jax.ShapeDtypeStruct((N_PAD, C_PAD), jnp.float32),
                   jax.ShapeDtypeStruct((N_PAD, C_PAD), jnp.float32)],
    )(acc1, xw1, degp, b1, w2p)


def _stage3(acc2, xw2, degp, b2p):
    return pl.pallas_call(
        _stage3_body,
        grid=(G_TC,),
        in_specs=[_part_spec(C_PAD), _row_spec(C_PAD), _part_spec(128),
                  _full_spec(1, C_PAD)],
        out_specs=_row_spec(C_OUT),
        out_shape=jax.ShapeDtypeStruct((N_PAD, C_OUT), jnp.float32),
    )(acc2, xw2, degp, b2p)


def kernel(x, edge_index, W1, b1, W2, b2):
    idt = edge_index.dtype
    pad_idx = jnp.full((E_PAD - E,), N, dtype=idt)
    srcp = jnp.concatenate([edge_index[0], pad_idx]).reshape(E_PAD // 128, 128)
    dstp = jnp.concatenate([edge_index[1], pad_idx]).reshape(E_PAD // 128, 128)
    srcp = srcp.astype(jnp.int32)
    dstp = dstp.astype(jnp.int32)

    x_p = jnp.pad(x, ((0, N_PAD - N), (0, 0)))
    w2p = jnp.pad(W2, ((0, 0), (0, C_PAD - C_OUT)))
    b1r = b1.reshape(1, HID)
    b2p = jnp.pad(b2, (0, C_PAD - C_OUT)).reshape(1, C_PAD)
    zeros128 = jnp.zeros((N_PAD, HID), jnp.float32)
    ones128 = jnp.ones((128, 128), jnp.float32)

    degp = _make_deg()(dstp, zeros128, ones128)
    y1, xw1 = _stage1(x_p, W1, degp)
    acc1 = _make_agg()(y1, srcp, dstp, zeros128)
    y2, xw2 = _stage2(acc1, xw1, degp, b1r, w2p)
    acc2 = _make_agg()(y2, srcp, dstp, zeros128)
    outp = _stage3(acc2, xw2, degp, b2p)
    return outp[:N]
